# baseline (speedup 1.0000x reference)
import functools

import jax
import jax.numpy as jnp
from jax import lax
from jax.experimental import pallas as pl
from jax.experimental.pallas import tpu as pltpu
from jax.experimental.pallas import tpu_sc as plsc

B, S, E = 4096, 200, 64
VOCAB = 100000
NW = 32
BT = B // NW
ET = E // 8
NBUF = 4
NG = S // NBUF

_mesh = plsc.VectorSubcoreMesh(core_axis_name="c", subcore_axis_name="s")


@functools.partial(
    pl.kernel,
    out_type=jax.ShapeDtypeStruct((S, ET, NW, 8, BT), jnp.float32),
    mesh=_mesh,
    scratch_types=(
        [pltpu.VMEM((S, BT), jnp.int32),
         pltpu.VMEM((S, E), jnp.float32),
         pltpu.VMEM((E,), jnp.float32)]
        + [pltpu.VMEM((BT, E), jnp.float32)] * NBUF
        + [pltpu.VMEM((E, BT + 1), jnp.float32)] * NBUF
        + [pltpu.SemaphoreType.DMA] * NBUF
        + [pltpu.SemaphoreType.DMA] * NBUF
    ),
    compiler_params=pltpu.CompilerParams(use_tc_tiling_on_sc=False, needs_layout_passes=False),
)
def _emb_kernel(xt_hbm, tok_hbm, pos_hbm, seg_hbm, out_hbm,
                xt_v, bias_v, seg_v, *rest):
    gbufs = rest[0:NBUF]
    wbufs = rest[NBUF:2 * NBUF]
    gsems = rest[2 * NBUF:3 * NBUF]
    osems = rest[3 * NBUF:4 * NBUF]

    wid = lax.axis_index("s") * 2 + lax.axis_index("c")

    pltpu.sync_copy(xt_hbm.at[wid], xt_v)
    pltpu.sync_copy(pos_hbm, bias_v)
    pltpu.sync_copy(seg_hbm.at[1], seg_v)

    def seg_body(r, carry):
        for u in range(E // 16):
            sl = pl.ds(u * 16, 16)
            bias_v[r, sl] = bias_v[r, sl] + seg_v[sl]
        return carry
    lax.fori_loop(0, S, seg_body, 0)

    lanes = lax.iota(jnp.int32, 16)
    e_idx = [u * 16 + lanes for u in range(E // 16)]

    def start_gather(s, b):
        pltpu.async_copy(tok_hbm.at[xt_v.at[s]], gbufs[b], gsems[b])

    def wait_gather(s, b):
        pltpu.make_async_copy(tok_hbm.at[xt_v.at[s]], gbufs[b],
                              gsems[b]).wait()

    def start_write(s, b):
        for et in range(ET):
            pltpu.async_copy(wbufs[b].at[pl.ds(et * 8, 8), pl.ds(0, BT)],
                             out_hbm.at[s, et, wid], osems[b])

    def wait_write(s, b):
        for et in range(ET):
            pltpu.make_async_copy(wbufs[b].at[pl.ds(et * 8, 8), pl.ds(0, BT)],
                                  out_hbm.at[s, et, wid], osems[b]).wait()

    for b in range(NBUF):
        start_gather(b, b)

    def round_body(g, carry):
        for b in range(NBUF):
            s = g * NBUF + b
            wait_gather(s, b)

            @pl.when(g > 0)
            def _():
                wait_write(s - NBUF, b)

            brow = [bias_v[s, pl.ds(u * 16, 16)] for u in range(E // 16)]

            @functools.partial(plsc.parallel_loop, 0, BT, carry=lanes * 0)
            def tr_body(r, rv, _b=b, _brow=brow):
                for u in range(E // 16):
                    val = gbufs[_b][r, pl.ds(u * 16, 16)] + _brow[u]
                    plsc.store_scatter(wbufs[_b], [e_idx[u], rv], val)
                return rv + 1

            start_write(s, b)

            @pl.when(g < NG - 1)
            def _():
                start_gather(s + NBUF, b)
        return carry
    lax.fori_loop(0, NG, round_body, 0)

    for b in range(NBUF):
        wait_write((NG - 1) * NBUF + b, b)


def kernel(x, token_table, pos_table, seg_table):
    xt = x.astype(jnp.int32).T.reshape(S, NW, BT).transpose(1, 0, 2)
    xt = jnp.remainder(xt + VOCAB, VOCAB)
    out = _emb_kernel(xt, token_table, pos_table, seg_table)
    out = out.reshape(S, ET, NW, 8, BT).transpose(2, 4, 0, 1, 3)
    return out.reshape(B, S, E)


# speedup vs baseline: 1.0177x; 1.0177x over previous
import functools

import jax
import jax.numpy as jnp
from jax import lax
from jax.experimental import pallas as pl
from jax.experimental.pallas import tpu as pltpu
from jax.experimental.pallas import tpu_sc as plsc

B, S, E = 4096, 200, 64
NW = 32
BT = B // NW
ET = E // 8
NBUF = 4
NG = S // NBUF

_mesh = plsc.VectorSubcoreMesh(core_axis_name="c", subcore_axis_name="s")


@functools.partial(
    pl.kernel,
    out_type=jax.ShapeDtypeStruct((S, ET, NW, 8, BT), jnp.float32),
    mesh=_mesh,
    scratch_types=(
        [pltpu.VMEM((S, BT), jnp.int32),
         pltpu.VMEM((S, E), jnp.float32),
         pltpu.VMEM((E,), jnp.float32)]
        + [pltpu.VMEM((BT, E), jnp.float32)] * NBUF
        + [pltpu.VMEM((E, BT + 1), jnp.float32)] * NBUF
        + [pltpu.SemaphoreType.DMA] * NBUF
        + [pltpu.SemaphoreType.DMA] * NBUF
    ),
    compiler_params=pltpu.CompilerParams(use_tc_tiling_on_sc=False, needs_layout_passes=False),
)
def _emb_kernel(xt_hbm, tok_hbm, pos_hbm, seg_hbm, out_hbm,
                xt_v, bias_v, seg_v, *rest):
    gbufs = rest[0:NBUF]
    wbufs = rest[NBUF:2 * NBUF]
    gsems = rest[2 * NBUF:3 * NBUF]
    osems = rest[3 * NBUF:4 * NBUF]

    wid = lax.axis_index("s") * 2 + lax.axis_index("c")

    pltpu.sync_copy(xt_hbm.at[wid], xt_v)
    pltpu.sync_copy(pos_hbm, bias_v)
    pltpu.sync_copy(seg_hbm.at[1], seg_v)


    lanes = lax.iota(jnp.int32, 16)
    e_idx = [u * 16 + lanes for u in range(E // 16)]

    def start_gather(s, b):
        pltpu.async_copy(tok_hbm.at[xt_v.at[s]], gbufs[b], gsems[b])

    def wait_gather(s, b):
        pltpu.make_async_copy(tok_hbm.at[xt_v.at[s]], gbufs[b],
                              gsems[b]).wait()

    def start_write(s, b):
        for et in range(ET):
            pltpu.async_copy(wbufs[b].at[pl.ds(et * 8, 8), pl.ds(0, BT)],
                             out_hbm.at[s, et, wid], osems[b])

    def wait_write(s, b):
        for et in range(ET):
            pltpu.make_async_copy(wbufs[b].at[pl.ds(et * 8, 8), pl.ds(0, BT)],
                                  out_hbm.at[s, et, wid], osems[b]).wait()

    for b in range(NBUF):
        start_gather(b, b)

    @functools.partial(plsc.parallel_loop, 0, S)
    def seg_body(r):
        for u in range(E // 16):
            sl = pl.ds(u * 16, 16)
            bias_v[r, sl] = bias_v[r, sl] + seg_v[sl]

    def round_body(g, carry):
        for b in range(NBUF):
            s = g * NBUF + b
            wait_gather(s, b)

            @pl.when(g > 0)
            def _():
                wait_write(s - NBUF, b)

            brow = [bias_v[s, pl.ds(u * 16, 16)] for u in range(E // 16)]

            @functools.partial(plsc.parallel_loop, 0, BT, carry=lanes * 0)
            def tr_body(r, rv, _b=b, _brow=brow):
                for u in range(E // 16):
                    val = gbufs[_b][r, pl.ds(u * 16, 16)] + _brow[u]
                    plsc.store_scatter(wbufs[_b], [e_idx[u], rv], val)
                return rv + 1

            start_write(s, b)

            @pl.when(g < NG - 1)
            def _():
                start_gather(s + NBUF, b)
        return carry
    lax.fori_loop(0, NG, round_body, 0)

    for b in range(NBUF):
        wait_write((NG - 1) * NBUF + b, b)


def kernel(x, token_table, pos_table, seg_table):
    xt = x.astype(jnp.int32).T.reshape(S, NW, BT).transpose(1, 0, 2)
    out = _emb_kernel(xt, token_table, pos_table, seg_table)
    out = out.reshape(S, ET, NW, 8, BT).transpose(2, 4, 0, 1, 3)
    return out.reshape(B, S, E)
